# baseline (speedup 1.0000x reference)
import math

import jax
import jax.numpy as jnp
from jax import lax
from jax.experimental import pallas as pl
from jax.experimental.pallas import tpu as pltpu
from jax.experimental.pallas import tpu_sc as plsc

MAX_SEQ_LEN = 512
EMB_SIZE = 128
VOCAB = 100000
BATCH = 4096
SEQ = 200

NUM_CORES = 2
NUM_SUBCORES = 16
NW = NUM_CORES * NUM_SUBCORES
BBLK = BATCH // NW
NCHUNK = SEQ
NBUF = 6
PREF = 4
SCALE = math.sqrt(float(EMB_SIZE))
ND = EMB_SIZE // 16


def _emb_kernel(srcT_hbm, table_hbm, pe_hbm, out_hbm,
                idx_v, rows_v, pe_ring, gsem, osem):
    wid = lax.axis_index("s") * NUM_CORES + lax.axis_index("c")
    b0 = wid * BBLK

    pltpu.sync_copy(srcT_hbm.at[wid], idx_v)

    def start_gather(p, slot):
        pltpu.async_copy(table_hbm.at[idx_v.at[p]], rows_v.at[slot],
                         gsem.at[slot])
        pltpu.async_copy(pe_hbm.at[pl.ds(p, 1)], pe_ring.at[slot],
                         gsem.at[slot])

    def wait_gather(p, slot):
        pltpu.make_async_copy(table_hbm.at[idx_v.at[p]], rows_v.at[slot],
                              gsem.at[slot]).wait()
        pltpu.make_async_copy(pe_hbm.at[pl.ds(p, 1)], pe_ring.at[slot],
                              gsem.at[slot]).wait()

    def out_dst(p):
        return out_hbm.at[pl.ds(b0, BBLK), p]

    def start_out(p, slot):
        pltpu.async_copy(rows_v.at[slot], out_dst(p), osem.at[slot])

    def wait_out(p, slot):
        pltpu.make_async_copy(rows_v.at[slot], out_dst(p),
                              osem.at[slot]).wait()

    def compute(slot):
        pev = [pe_ring[slot, 0, pl.ds(d * 16, 16)] for d in range(ND)]

        @plsc.parallel_loop(0, BBLK, unroll=2)
        def _row(r):
            row = [rows_v[slot, r, pl.ds(d * 16, 16)] for d in range(ND)]
            for d in range(ND):
                rows_v[slot, r, pl.ds(d * 16, 16)] = (row[d] * SCALE
                                                      + pev[d])

    def do_chunk(p, slot):
        wait_gather(p, slot)
        compute(slot)
        start_out(p, slot)

    def prefetch(p, slot):
        pslot = (slot + PREF) % NBUF
        if isinstance(p, int):
            if 0 <= p + PREF - NBUF:
                wait_out(p + PREF - NBUF, pslot)
            if p + PREF < NCHUNK:
                start_gather(p + PREF, pslot)
        else:
            wait_out(p + PREF - NBUF, pslot)
            start_gather(p + PREF, pslot)

    for k in range(PREF):
        start_gather(k, k)

    for p in range(NBUF):
        do_chunk(p, p)
        prefetch(p, p)

    MAIN_END = NBUF + ((NCHUNK - 2 * NBUF) // NBUF) * NBUF

    @pl.loop(NBUF, MAIN_END, step=NBUF)
    def _super(pp):
        for k in range(NBUF):
            do_chunk(pp + k, k)
            prefetch(pp + k, k)

    for p in range(MAIN_END, NCHUNK):
        do_chunk(p, p % NBUF)
        prefetch(p, p % NBUF)

    for p in range(NCHUNK - (NBUF - PREF), NCHUNK):
        wait_out(p, p % NBUF)


@jax.jit
def _run(src, table, pe2):
    srcT = jnp.transpose(src.reshape(NW, BBLK, SEQ), (0, 2, 1))
    mesh = plsc.VectorSubcoreMesh(core_axis_name="c", subcore_axis_name="s")
    f = pl.kernel(
        _emb_kernel,
        out_type=jax.ShapeDtypeStruct((BATCH, SEQ, EMB_SIZE), jnp.float32),
        mesh=mesh,
        scratch_types=[
            pltpu.VMEM((SEQ, BBLK), jnp.int32),
            pltpu.VMEM((NBUF, BBLK, EMB_SIZE), jnp.float32),
            pltpu.VMEM((NBUF, 1, EMB_SIZE), jnp.float32),
            pltpu.SemaphoreType.DMA((NBUF,)),
            pltpu.SemaphoreType.DMA((NBUF,)),
        ],
    )
    return f(srcT, table, pe2)


def kernel(src, step, table, pe):
    del step
    return _run(src, table, pe[:SEQ, 0, :])
